# baseline (speedup 1.0000x reference)
import functools

import jax
import jax.numpy as jnp
from jax import lax
from jax.experimental import pallas as pl
from jax.experimental.pallas import tpu as pltpu
from jax.experimental.pallas import tpu_sc as plsc

EMB_DIM = 64
SUB = 4
SUB_W = EMB_DIM // SUB
NUM_WORKERS = 32
CHUNK = 800


def _gather_body(n_chunks, rows_per_worker, table_hbm, idx_hbm, out_hbm,
                 i0, i1, rows0, rows1, sg0, sg1, so0, so1):
    cid = lax.axis_index("c")
    sid = lax.axis_index("s")
    wid = sid * 2 + cid
    base = wid * rows_per_worker

    idx = (i0, i1)
    rows = (rows0, rows1)
    sem_g = (sg0, sg1)
    sem_o = (so0, so1)

    def fire_gather(g, j):
        pltpu.sync_copy(idx_hbm.at[pl.ds(base + g * CHUNK, CHUNK)], idx[j])
        pltpu.async_copy(table_hbm.at[idx[j]], rows[j], sem_g[j])

    def wait_gather(j):
        pltpu.make_async_copy(table_hbm.at[idx[j]], rows[j], sem_g[j]).wait()

    def fire_out(g, j):
        pltpu.async_copy(rows[j], out_hbm.at[pl.ds(base + g * CHUNK, CHUNK)],
                         sem_o[j])

    def wait_out(j):
        pltpu.make_async_copy(rows[j], out_hbm.at[pl.ds(base, CHUNK)],
                              sem_o[j]).wait()

    fire_gather(0, 0)

    def pair(i, carry):
        for j in (0, 1):
            g = 2 * i + j

            @pl.when(g > 0)
            def _():
                wait_out(1 - j)

            @pl.when(g < n_chunks - 1)
            def _():
                fire_gather(g + 1, 1 - j)

            wait_gather(j)
            fire_out(g, j)
        return carry

    lax.fori_loop(0, n_chunks // 2, pair, 0, unroll=False)
    wait_out((n_chunks - 1) % 2)


def kernel(embeddings, indices):
    flat_idx = indices.reshape(-1).astype(jnp.int32)
    total = flat_idx.shape[0]
    rows_per_worker = total // NUM_WORKERS
    n_chunks = rows_per_worker // CHUNK
    assert rows_per_worker * NUM_WORKERS == total
    assert n_chunks * CHUNK == rows_per_worker and n_chunks % 2 == 0

    table3 = embeddings.reshape(embeddings.shape[0], SUB, SUB_W)

    mesh = plsc.VectorSubcoreMesh(core_axis_name="c", subcore_axis_name="s")
    grab = pl.kernel(
        functools.partial(_gather_body, n_chunks, rows_per_worker),
        out_type=jax.ShapeDtypeStruct((total, SUB, SUB_W), jnp.float32),
        mesh=mesh,
        scratch_types=[
            pltpu.VMEM((CHUNK,), jnp.int32),
            pltpu.VMEM((CHUNK,), jnp.int32),
            pltpu.VMEM((CHUNK, SUB, SUB_W), jnp.float32),
            pltpu.VMEM((CHUNK, SUB, SUB_W), jnp.float32),
            pltpu.SemaphoreType.DMA,
            pltpu.SemaphoreType.DMA,
            pltpu.SemaphoreType.DMA,
            pltpu.SemaphoreType.DMA,
        ],
        compiler_params=pltpu.CompilerParams(use_tc_tiling_on_sc=False),
    )
    out = grab(table3, flat_idx)
    return out.reshape(indices.shape + (EMB_DIM,))


# speedup vs baseline: 1.9568x; 1.9568x over previous
import functools

import jax
import jax.numpy as jnp
from jax import lax
from jax.experimental import pallas as pl
from jax.experimental.pallas import tpu as pltpu
from jax.experimental.pallas import tpu_sc as plsc

EMB_DIM = 64
SUB = 4
SUB_W = EMB_DIM // SUB
LANES = 16
NUM_WORKERS = 32
CHUNK = 800


def _gather_body(n_chunks, rows_per_worker, table_hbm, idx_hbm, out_hbm,
                 r0, r1, e0, e1, rows0, rows1, sg0, sg1, so0, so1):
    cid = lax.axis_index("c")
    sid = lax.axis_index("s")
    wid = sid * 2 + cid
    base = wid * rows_per_worker

    raw = (r0, r1)
    idx4 = (e0, e1)
    rows = (rows0, rows1)
    sem_g = (sg0, sg1)
    sem_o = (so0, so1)

    lane4 = lax.iota(jnp.int32, LANES) * SUB

    def fire_gather(g, j):
        pltpu.sync_copy(idx_hbm.at[pl.ds(base + g * CHUNK, CHUNK)], raw[j])

        def expand(i, carry):
            v4 = raw[j][pl.ds(i * LANES, LANES)] * SUB
            pos = lane4 + i * (LANES * SUB)
            for k in range(SUB):
                plsc.store_scatter(idx4[j], [pos + k], v4 + k)
            return carry

        lax.fori_loop(0, CHUNK // LANES, expand, 0, unroll=2)
        pltpu.async_copy(table_hbm.at[idx4[j]], rows[j], sem_g[j])

    def wait_gather(j):
        pltpu.make_async_copy(table_hbm.at[idx4[j]], rows[j], sem_g[j]).wait()

    def fire_out(g, j):
        pltpu.async_copy(rows[j],
                         out_hbm.at[pl.ds((base + g * CHUNK) * SUB,
                                          CHUNK * SUB)],
                         sem_o[j])

    def wait_out(j):
        pltpu.make_async_copy(rows[j],
                              out_hbm.at[pl.ds(base * SUB, CHUNK * SUB)],
                              sem_o[j]).wait()

    fire_gather(0, 0)

    def pair(i, carry):
        for j in (0, 1):
            g = 2 * i + j

            @pl.when(g > 0)
            def _():
                wait_out(1 - j)

            @pl.when(g < n_chunks - 1)
            def _():
                fire_gather(g + 1, 1 - j)

            wait_gather(j)
            fire_out(g, j)
        return carry

    lax.fori_loop(0, n_chunks // 2, pair, 0, unroll=False)
    wait_out((n_chunks - 1) % 2)


def kernel(embeddings, indices):
    flat_idx = indices.reshape(-1).astype(jnp.int32)
    total = flat_idx.shape[0]
    rows_per_worker = total // NUM_WORKERS
    n_chunks = rows_per_worker // CHUNK
    assert rows_per_worker * NUM_WORKERS == total
    assert n_chunks * CHUNK == rows_per_worker and n_chunks % 2 == 0

    table4 = embeddings.reshape(embeddings.shape[0] * SUB, SUB_W)

    mesh = plsc.VectorSubcoreMesh(core_axis_name="c", subcore_axis_name="s")
    grab = pl.kernel(
        functools.partial(_gather_body, n_chunks, rows_per_worker),
        out_type=jax.ShapeDtypeStruct((total * SUB, SUB_W), jnp.float32),
        mesh=mesh,
        scratch_types=[
            pltpu.VMEM((CHUNK,), jnp.int32),
            pltpu.VMEM((CHUNK,), jnp.int32),
            pltpu.VMEM((CHUNK * SUB,), jnp.int32),
            pltpu.VMEM((CHUNK * SUB,), jnp.int32),
            pltpu.VMEM((CHUNK * SUB, SUB_W), jnp.float32),
            pltpu.VMEM((CHUNK * SUB, SUB_W), jnp.float32),
            pltpu.SemaphoreType.DMA,
            pltpu.SemaphoreType.DMA,
            pltpu.SemaphoreType.DMA,
            pltpu.SemaphoreType.DMA,
        ],
        compiler_params=pltpu.CompilerParams(use_tc_tiling_on_sc=False,
                                             needs_layout_passes=False),
    )
    out = grab(table4, flat_idx)
    return out.reshape(indices.shape + (EMB_DIM,))
